# baseline (speedup 1.0000x reference)
import jax
import jax.numpy as jnp
from jax.experimental import pallas as pl
from jax.experimental.pallas import tpu as pltpu

_X = 48
_H = 64
_B = 8
_NODES = 8
_TD = 100
_NH = 4
_F = 64
_SLOPE = 0.2
_CL = 16


def _lstm_body(g_all):
    rows = _NODES * g_all

    def body(x_ref, c_ref, t_ref, wiou_ref, uiou_ref, ufw_ref, wtype_ref,
             wfc_ref, biou_ref, ufb_ref, hr_ref, fs_ref):
        f32 = jnp.float32
        sig = lambda v: 0.5 * jnp.tanh(0.5 * v) + 0.5
        b_iou = biou_ref[...]
        u_f_b = ufb_ref[...]

        x2 = jnp.transpose(x_ref[0], (0, 2, 1)).reshape(rows, _X)
        c2 = jnp.transpose(c_ref[0], (0, 2, 1)).reshape(rows, _H)

        iou = jnp.dot(x2, wiou_ref[...],
                      preferred_element_type=f32) + b_iou
        io = sig(iou[:, 0:2 * _H])
        u_g = jnp.tanh(iou[:, 2 * _H:3 * _H])
        c_all = io[:, 0:_H] * u_g + c2
        h_all = io[:, _H:2 * _H] * jnp.tanh(c_all)
        f_g = sig(jnp.dot(h_all, ufw_ref[...],
                          preferred_element_type=f32) + u_f_b)
        fc = f_g * c_all

        rid = jax.lax.broadcasted_iota(jnp.int32, (rows, 1), 0)
        leaf = (rid // g_all != 0).astype(f32)
        hc = jnp.concatenate([h_all, fc], axis=1) * leaf
        red = jnp.sum(hc.reshape(_NODES, g_all, 2 * _H), axis=0)

        iou_r = jnp.dot(red[:, 0:_H], uiou_ref[...],
                        preferred_element_type=f32) + b_iou
        c_root = (sig(iou_r[:, 0:_H]) *
                  jnp.tanh(iou_r[:, 2 * _H:3 * _H]) + red[:, _H:2 * _H])
        hr_ref[0] = sig(iou_r[:, _H:2 * _H]) * jnp.tanh(c_root)

        wtf = jnp.dot(wtype_ref[...], wfc_ref[...],
                      preferred_element_type=f32)
        tb = t_ref[:, pl.ds(pl.program_id(0), 1), :]
        t2 = jnp.transpose(tb.reshape(_TD, g_all), (1, 0))
        fs_ref[0] = jnp.dot(t2, wtf,
                            preferred_element_type=f32)

    return body


def _gat_body(hr_ref, fs_ref, a_ref, al_ref, ar_ref, bg_ref, w1_ref, w2_ref,
              w3_ref, w4_ref, b1_ref, b2_ref, b3_ref, b4_ref, out_ref):
    f32 = jnp.float32
    nc = _B * _CL
    cdim = (((1,), (1,)), ((), ()))

    hr4 = jax.lax.broadcasted_iota(jnp.int32, (_NH, _NH * _F), 0)
    hc4 = jax.lax.broadcasted_iota(jnp.int32, (_NH, _NH * _F), 1)
    hmask = (hc4 // _F == hr4).astype(f32)
    al4 = hmask * al_ref[...]
    ar4 = hmask * ar_ref[...]

    wmt = jax.lax.dot_general(w4_ref[...], w3_ref[...],
                              (((0,), (1,)), ((), ())),
                              preferred_element_type=f32)
    wmt = jax.lax.dot_general(wmt, w2_ref[...], cdim,
                              preferred_element_type=f32)
    wmt = jax.lax.dot_general(wmt, w1_ref[...], cdim,
                              preferred_element_type=f32)
    wm8 = jnp.concatenate(
        [wmt, jnp.zeros((6, 2 * _H), f32)], axis=0)
    bm = (jnp.dot(jnp.dot(jnp.dot(b1_ref[...], w2_ref[...]) + b2_ref[...],
                          w3_ref[...]) + b3_ref[...],
                  w4_ref[...]) + b4_ref[...])
    er8 = jax.lax.broadcasted_iota(jnp.int32, (8, 2), 0)
    ec8 = jax.lax.broadcasted_iota(jnp.int32, (8, 2), 1)
    eye82 = (er8 == ec8).astype(f32)
    bm_col = jax.lax.dot_general(eye82, bm, cdim,
                                 preferred_element_type=f32)

    hr = hr_ref[...].reshape(nc, _H)
    fs = fs_ref[...].reshape(nc, 2 * _H * 2)
    am = (a_ref[...].reshape(nc, _B) > 0).astype(f32)

    el = jax.lax.dot_general(al4, fs, cdim,
                             preferred_element_type=f32)
    er = jax.lax.dot_general(fs, ar4, cdim,
                             preferred_element_type=f32)

    tr = jax.lax.broadcasted_iota(jnp.int32, (_B, nc), 0)
    tc = jax.lax.broadcasted_iota(jnp.int32, (_B, nc), 1)
    tsel = (tc // _CL == tr).astype(f32)
    tiled = jnp.dot(am, tsel, preferred_element_type=f32)
    ri = jax.lax.broadcasted_iota(jnp.int32, (nc, nc), 0)
    ci = jax.lax.broadcasted_iota(jnp.int32, (nc, nc), 1)
    mask = jnp.where((ri % _CL) == (ci % _CL), tiled, 0.0)

    acc = None
    for h in range(_NH):
        e = el[h:h + 1, :] + er[:, h:h + 1]
        e = jnp.maximum(e, _SLOPE * e)
        e = jnp.where(mask > 0, e, -1e30)
        m = jnp.max(e, axis=1, keepdims=True)
        p = jnp.exp(e - m) * mask
        d = jnp.sum(p, axis=1, keepdims=True)
        att = p / jnp.maximum(d, 1e-30)
        r = jnp.dot(att, hr, preferred_element_type=f32)
        r = jnp.maximum(r + bg_ref[0:1, h * _F:(h + 1) * _F], 0.0)
        acc = r if acc is None else acc + r

    cat = jnp.concatenate([acc * (1.0 / _NH), hr], axis=1)
    out_ref[...] = jax.lax.dot_general(
        wm8, cat, cdim, preferred_element_type=f32) + bm_col


def kernel(w_iou, b_iou, u_iou, u_f_w, u_f_b, w_type, w_fc, attn_l, attn_r,
           bias_gat, w1, b1, w2, b2, w3, b3, w4, b4,
           x_ast, h0, c0, cfg_type, adj):
    del h0
    f32 = jnp.float32
    g_all = x_ast.shape[0]
    nc = _B * _CL
    chunks = g_all // _CL

    hbm = lambda v: pltpu.with_memory_space_constraint(
        v, pltpu.MemorySpace.HBM)
    xb = hbm(jnp.transpose(x_ast, (1, 2, 3, 0)))
    cb = hbm(jnp.transpose(c0, (1, 2, 3, 0)))
    tb = hbm(jnp.transpose(cfg_type, (2, 1, 0)))
    at = jnp.transpose(adj, (1, 0, 2))

    h_root, fsrc = pl.pallas_call(
        _lstm_body(g_all),
        out_shape=[
            jax.ShapeDtypeStruct((_B, g_all, _H), f32),
            jax.ShapeDtypeStruct((_B, g_all, 2 * _H * 2), f32),
        ],
        grid=(_B,),
        in_specs=[
            pl.BlockSpec((1, _NODES, _X, g_all), lambda b: (b, 0, 0, 0)),
            pl.BlockSpec((1, _NODES, _H, g_all), lambda b: (b, 0, 0, 0)),
            pl.BlockSpec((_TD, _B, g_all), lambda b: (0, 0, 0)),
            pl.BlockSpec(w_iou.shape, lambda b: (0, 0)),
            pl.BlockSpec(u_iou.shape, lambda b: (0, 0)),
            pl.BlockSpec(u_f_w.shape, lambda b: (0, 0)),
            pl.BlockSpec(w_type.shape, lambda b: (0, 0)),
            pl.BlockSpec(w_fc.shape, lambda b: (0, 0)),
            pl.BlockSpec(b_iou.shape, lambda b: (0, 0)),
            pl.BlockSpec(u_f_b.shape, lambda b: (0, 0)),
        ],
        out_specs=[
            pl.BlockSpec((1, g_all, _H), lambda b: (b, 0, 0)),
            pl.BlockSpec((1, g_all, 2 * _H * 2), lambda b: (b, 0, 0)),
        ],
        compiler_params=pltpu.CompilerParams(
            dimension_semantics=("parallel",)),
        cost_estimate=pl.CostEstimate(
            flops=2_400_000 * g_all, transcendentals=18_000 * g_all,
            bytes_accessed=33_000 * g_all),
    )(xb, cb, tb, w_iou, u_iou, u_f_w, w_type, w_fc, b_iou, u_f_b)

    out = pl.pallas_call(
        _gat_body,
        out_shape=jax.ShapeDtypeStruct((8, g_all * _B), f32),
        grid=(chunks,),
        in_specs=[
            pl.BlockSpec((_B, _CL, _H), lambda c: (0, c, 0)),
            pl.BlockSpec((_B, _CL, 2 * _H * 2), lambda c: (0, c, 0)),
            pl.BlockSpec((_B, _CL, _B), lambda c: (0, c, 0)),
            pl.BlockSpec(attn_l.shape, lambda c: (0, 0)),
            pl.BlockSpec(attn_r.shape, lambda c: (0, 0)),
            pl.BlockSpec(bias_gat.shape, lambda c: (0, 0)),
            pl.BlockSpec(w1.shape, lambda c: (0, 0)),
            pl.BlockSpec(w2.shape, lambda c: (0, 0)),
            pl.BlockSpec(w3.shape, lambda c: (0, 0)),
            pl.BlockSpec(w4.shape, lambda c: (0, 0)),
            pl.BlockSpec(b1.shape, lambda c: (0, 0)),
            pl.BlockSpec(b2.shape, lambda c: (0, 0)),
            pl.BlockSpec(b3.shape, lambda c: (0, 0)),
            pl.BlockSpec(b4.shape, lambda c: (0, 0)),
        ],
        out_specs=pl.BlockSpec((8, nc), lambda c: (0, c)),
        compiler_params=pltpu.CompilerParams(
            dimension_semantics=("parallel",)),
        cost_estimate=pl.CostEstimate(
            flops=600_000 * g_all, transcendentals=3_000 * g_all,
            bytes_accessed=12_000 * g_all),
    )(h_root, fsrc, at, attn_l, attn_r, bias_gat,
      w1, w2, w3, w4, b1, b2, b3, b4)

    o = out.reshape(8, chunks, _B, _CL)
    o = jnp.transpose(o, (1, 3, 2, 0))
    return o.reshape(g_all, _B, 8)[:, :, 0:2]


# speedup vs baseline: 1.1691x; 1.1691x over previous
import jax
import jax.numpy as jnp
from jax.experimental import pallas as pl
from jax.experimental.pallas import tpu as pltpu

_X = 48
_H = 64
_B = 8
_NODES = 8
_TD = 100
_NH = 4
_F = 64
_SLOPE = 0.2
_CL = 32


def _lstm_body(g_all):
    rows = _NODES * g_all

    def body(x_ref, c_ref, t_ref, wiou_ref, uiou_ref, ufw_ref, wtype_ref,
             wfc_ref, biou_ref, ufb_ref, hr_ref, fs_ref):
        f32 = jnp.float32
        sig = lambda v: 0.5 * jnp.tanh(0.5 * v) + 0.5
        b_iou = biou_ref[...]
        u_f_b = ufb_ref[...]

        x2 = jnp.transpose(x_ref[0], (0, 2, 1)).reshape(rows, _X)
        c2 = jnp.transpose(c_ref[0], (0, 2, 1)).reshape(rows, _H)

        iou = jnp.dot(x2, wiou_ref[...],
                      preferred_element_type=f32) + b_iou
        io = sig(iou[:, 0:2 * _H])
        u_g = jnp.tanh(iou[:, 2 * _H:3 * _H])
        c_all = io[:, 0:_H] * u_g + c2
        h_all = io[:, _H:2 * _H] * jnp.tanh(c_all)
        f_g = sig(jnp.dot(h_all, ufw_ref[...],
                          preferred_element_type=f32) + u_f_b)
        fc = f_g * c_all

        rid = jax.lax.broadcasted_iota(jnp.int32, (rows, 1), 0)
        leaf = (rid // g_all != 0).astype(f32)
        hc = jnp.concatenate([h_all, fc], axis=1) * leaf
        red = jnp.sum(hc.reshape(_NODES, g_all, 2 * _H), axis=0)

        iou_r = jnp.dot(red[:, 0:_H], uiou_ref[...],
                        preferred_element_type=f32) + b_iou
        c_root = (sig(iou_r[:, 0:_H]) *
                  jnp.tanh(iou_r[:, 2 * _H:3 * _H]) + red[:, _H:2 * _H])
        hr_ref[0] = sig(iou_r[:, _H:2 * _H]) * jnp.tanh(c_root)

        wtf = jnp.dot(wtype_ref[...], wfc_ref[...],
                      preferred_element_type=f32)
        tb = t_ref[:, pl.ds(pl.program_id(0), 1), :]
        t2 = jnp.transpose(tb.reshape(_TD, g_all), (1, 0))
        fs_ref[0] = jnp.dot(t2, wtf,
                            preferred_element_type=f32)

    return body


def _gat_body(hr_ref, fs_ref, a_ref, al_ref, ar_ref, bg_ref, w1_ref, w2_ref,
              w3_ref, w4_ref, b1_ref, b2_ref, b3_ref, b4_ref, out_ref):
    f32 = jnp.float32
    nc = _B * _CL
    cdim = (((1,), (1,)), ((), ()))

    hr4 = jax.lax.broadcasted_iota(jnp.int32, (_NH, _NH * _F), 0)
    hc4 = jax.lax.broadcasted_iota(jnp.int32, (_NH, _NH * _F), 1)
    hmask = (hc4 // _F == hr4).astype(f32)
    al4 = hmask * al_ref[...]
    ar4 = hmask * ar_ref[...]

    wmt = jax.lax.dot_general(w4_ref[...], w3_ref[...],
                              (((0,), (1,)), ((), ())),
                              preferred_element_type=f32)
    wmt = jax.lax.dot_general(wmt, w2_ref[...], cdim,
                              preferred_element_type=f32)
    wmt = jax.lax.dot_general(wmt, w1_ref[...], cdim,
                              preferred_element_type=f32)
    wm8 = jnp.concatenate(
        [wmt, jnp.zeros((6, 2 * _H), f32)], axis=0)
    bm = (jnp.dot(jnp.dot(jnp.dot(b1_ref[...], w2_ref[...]) + b2_ref[...],
                          w3_ref[...]) + b3_ref[...],
                  w4_ref[...]) + b4_ref[...])
    er8 = jax.lax.broadcasted_iota(jnp.int32, (8, 2), 0)
    ec8 = jax.lax.broadcasted_iota(jnp.int32, (8, 2), 1)
    eye82 = (er8 == ec8).astype(f32)
    bm_col = jax.lax.dot_general(eye82, bm, cdim,
                                 preferred_element_type=f32)

    hr = hr_ref[...].reshape(nc, _H)
    fs = fs_ref[...].reshape(nc, 2 * _H * 2)
    am = (a_ref[...].reshape(nc, _B) > 0).astype(f32)

    el = jax.lax.dot_general(al4, fs, cdim,
                             preferred_element_type=f32)
    er = jax.lax.dot_general(fs, ar4, cdim,
                             preferred_element_type=f32)

    tr = jax.lax.broadcasted_iota(jnp.int32, (_B, nc), 0)
    tc = jax.lax.broadcasted_iota(jnp.int32, (_B, nc), 1)
    tsel = (tc // _CL == tr).astype(f32)
    tiled = jnp.dot(am, tsel, preferred_element_type=f32)
    ri = jax.lax.broadcasted_iota(jnp.int32, (nc, nc), 0)
    ci = jax.lax.broadcasted_iota(jnp.int32, (nc, nc), 1)
    mask = jnp.where((ri % _CL) == (ci % _CL), tiled, 0.0)

    acc = None
    for h in range(_NH):
        e = el[h:h + 1, :] + er[:, h:h + 1]
        e = jnp.maximum(e, _SLOPE * e)
        e = jnp.where(mask > 0, e, -1e30)
        m = jnp.max(e, axis=1, keepdims=True)
        p = jnp.exp(e - m) * mask
        d = jnp.sum(p, axis=1, keepdims=True)
        att = p / jnp.maximum(d, 1e-30)
        r = jnp.dot(att, hr, preferred_element_type=f32)
        r = jnp.maximum(r + bg_ref[0:1, h * _F:(h + 1) * _F], 0.0)
        acc = r if acc is None else acc + r

    cat = jnp.concatenate([acc * (1.0 / _NH), hr], axis=1)
    out_ref[...] = jax.lax.dot_general(
        wm8, cat, cdim, preferred_element_type=f32) + bm_col


def kernel(w_iou, b_iou, u_iou, u_f_w, u_f_b, w_type, w_fc, attn_l, attn_r,
           bias_gat, w1, b1, w2, b2, w3, b3, w4, b4,
           x_ast, h0, c0, cfg_type, adj):
    del h0
    f32 = jnp.float32
    g_all = x_ast.shape[0]
    nc = _B * _CL
    chunks = g_all // _CL

    hbm = lambda v: pltpu.with_memory_space_constraint(
        v, pltpu.MemorySpace.HBM)
    xb = hbm(jnp.transpose(x_ast, (1, 2, 3, 0)))
    cb = hbm(jnp.transpose(c0, (1, 2, 3, 0)))
    tb = hbm(jnp.transpose(cfg_type, (2, 1, 0)))
    at = jnp.transpose(adj, (1, 0, 2))

    h_root, fsrc = pl.pallas_call(
        _lstm_body(g_all),
        out_shape=[
            jax.ShapeDtypeStruct((_B, g_all, _H), f32),
            jax.ShapeDtypeStruct((_B, g_all, 2 * _H * 2), f32),
        ],
        grid=(_B,),
        in_specs=[
            pl.BlockSpec((1, _NODES, _X, g_all), lambda b: (b, 0, 0, 0)),
            pl.BlockSpec((1, _NODES, _H, g_all), lambda b: (b, 0, 0, 0)),
            pl.BlockSpec((_TD, _B, g_all), lambda b: (0, 0, 0)),
            pl.BlockSpec(w_iou.shape, lambda b: (0, 0)),
            pl.BlockSpec(u_iou.shape, lambda b: (0, 0)),
            pl.BlockSpec(u_f_w.shape, lambda b: (0, 0)),
            pl.BlockSpec(w_type.shape, lambda b: (0, 0)),
            pl.BlockSpec(w_fc.shape, lambda b: (0, 0)),
            pl.BlockSpec(b_iou.shape, lambda b: (0, 0)),
            pl.BlockSpec(u_f_b.shape, lambda b: (0, 0)),
        ],
        out_specs=[
            pl.BlockSpec((1, g_all, _H), lambda b: (b, 0, 0)),
            pl.BlockSpec((1, g_all, 2 * _H * 2), lambda b: (b, 0, 0)),
        ],
        compiler_params=pltpu.CompilerParams(
            dimension_semantics=("parallel",)),
        cost_estimate=pl.CostEstimate(
            flops=2_400_000 * g_all, transcendentals=18_000 * g_all,
            bytes_accessed=33_000 * g_all),
    )(xb, cb, tb, w_iou, u_iou, u_f_w, w_type, w_fc, b_iou, u_f_b)

    out = pl.pallas_call(
        _gat_body,
        out_shape=jax.ShapeDtypeStruct((8, g_all * _B), f32),
        grid=(chunks,),
        in_specs=[
            pl.BlockSpec((_B, _CL, _H), lambda c: (0, c, 0)),
            pl.BlockSpec((_B, _CL, 2 * _H * 2), lambda c: (0, c, 0)),
            pl.BlockSpec((_B, _CL, _B), lambda c: (0, c, 0)),
            pl.BlockSpec(attn_l.shape, lambda c: (0, 0)),
            pl.BlockSpec(attn_r.shape, lambda c: (0, 0)),
            pl.BlockSpec(bias_gat.shape, lambda c: (0, 0)),
            pl.BlockSpec(w1.shape, lambda c: (0, 0)),
            pl.BlockSpec(w2.shape, lambda c: (0, 0)),
            pl.BlockSpec(w3.shape, lambda c: (0, 0)),
            pl.BlockSpec(w4.shape, lambda c: (0, 0)),
            pl.BlockSpec(b1.shape, lambda c: (0, 0)),
            pl.BlockSpec(b2.shape, lambda c: (0, 0)),
            pl.BlockSpec(b3.shape, lambda c: (0, 0)),
            pl.BlockSpec(b4.shape, lambda c: (0, 0)),
        ],
        out_specs=pl.BlockSpec((8, nc), lambda c: (0, c)),
        compiler_params=pltpu.CompilerParams(
            dimension_semantics=("parallel",)),
        cost_estimate=pl.CostEstimate(
            flops=600_000 * g_all, transcendentals=3_000 * g_all,
            bytes_accessed=12_000 * g_all),
    )(h_root, fsrc, at, attn_l, attn_r, bias_gat,
      w1, w2, w3, w4, b1, b2, b3, b4)

    o = out.reshape(8, chunks, _B, _CL)
    o = jnp.transpose(o, (1, 3, 2, 0))
    return o.reshape(g_all, _B, 8)[:, :, 0:2]


# speedup vs baseline: 1.2418x; 1.0622x over previous
import jax
import jax.numpy as jnp
from jax.experimental import pallas as pl
from jax.experimental.pallas import tpu as pltpu

_X = 48
_H = 64
_B = 8
_NODES = 8
_TD = 100
_NH = 4
_F = 64
_SLOPE = 0.2
_CL = 32


def _lstm_body(g_all):
    rows = _NODES * g_all

    def body(x_ref, c_ref, t_ref, wiou_ref, uiou_ref, ufw_ref, wtf_ref,
             biou_ref, ufb_ref, hr_ref, fs_ref):
        f32 = jnp.float32
        sig = lambda v: 0.5 * jnp.tanh(0.5 * v) + 0.5
        b_iou = biou_ref[...]
        u_f_b = ufb_ref[...]

        x2 = jnp.transpose(x_ref[0], (0, 2, 1)).reshape(rows, _X)
        c2 = jnp.transpose(c_ref[0], (0, 2, 1)).reshape(rows, _H)

        iou = jnp.dot(x2, wiou_ref[...],
                      preferred_element_type=f32) + b_iou
        io = sig(iou[:, 0:2 * _H])
        u_g = jnp.tanh(iou[:, 2 * _H:3 * _H])
        c_all = io[:, 0:_H] * u_g + c2
        h_all = io[:, _H:2 * _H] * jnp.tanh(c_all)
        f_g = sig(jnp.dot(h_all, ufw_ref[...],
                          preferred_element_type=f32) + u_f_b)
        fc = f_g * c_all

        rid = jax.lax.broadcasted_iota(jnp.int32, (rows, 1), 0)
        leaf = (rid // g_all != 0).astype(f32)
        hc = jnp.concatenate([h_all, fc], axis=1) * leaf
        red = jnp.sum(hc.reshape(_NODES, g_all, 2 * _H), axis=0)

        iou_r = jnp.dot(red[:, 0:_H], uiou_ref[...],
                        preferred_element_type=f32) + b_iou
        c_root = (sig(iou_r[:, 0:_H]) *
                  jnp.tanh(iou_r[:, 2 * _H:3 * _H]) + red[:, _H:2 * _H])
        hr_ref[0] = sig(iou_r[:, _H:2 * _H]) * jnp.tanh(c_root)

        tb = t_ref[:, pl.ds(pl.program_id(0), 1), :]
        t2 = jnp.transpose(tb.reshape(_TD, g_all), (1, 0))
        fs_ref[0] = jnp.dot(t2, wtf_ref[...],
                            preferred_element_type=f32)

    return body


def _gat_body(hr_ref, fs_ref, a_ref, al_ref, ar_ref, bg_ref, wmt_ref,
              bm_ref, out_ref):
    f32 = jnp.float32
    nc = _B * _CL
    cdim = (((1,), (1,)), ((), ()))

    hr4 = jax.lax.broadcasted_iota(jnp.int32, (_NH, _NH * _F), 0)
    hc4 = jax.lax.broadcasted_iota(jnp.int32, (_NH, _NH * _F), 1)
    hmask = (hc4 // _F == hr4).astype(f32)
    al4 = hmask * al_ref[...]
    ar4 = hmask * ar_ref[...]

    wm8 = jnp.concatenate(
        [wmt_ref[...], jnp.zeros((6, 2 * _H), f32)], axis=0)
    er8 = jax.lax.broadcasted_iota(jnp.int32, (8, 2), 0)
    ec8 = jax.lax.broadcasted_iota(jnp.int32, (8, 2), 1)
    eye82 = (er8 == ec8).astype(f32)
    bm_col = jax.lax.dot_general(eye82, bm_ref[...], cdim,
                                 preferred_element_type=f32)

    hr = hr_ref[...].reshape(nc, _H)
    fs = fs_ref[...].reshape(nc, 2 * _H * 2)
    am = (a_ref[...].reshape(nc, _B) > 0).astype(f32)

    el = jax.lax.dot_general(al4, fs, cdim,
                             preferred_element_type=f32)
    er = jax.lax.dot_general(fs, ar4, cdim,
                             preferred_element_type=f32)

    tr = jax.lax.broadcasted_iota(jnp.int32, (_B, nc), 0)
    tc = jax.lax.broadcasted_iota(jnp.int32, (_B, nc), 1)
    tsel = (tc // _CL == tr).astype(f32)
    tiled = jnp.dot(am, tsel, preferred_element_type=f32)
    ri = jax.lax.broadcasted_iota(jnp.int32, (nc, nc), 0)
    ci = jax.lax.broadcasted_iota(jnp.int32, (nc, nc), 1)
    mask = jnp.where((ri % _CL) == (ci % _CL), tiled, 0.0)

    acc = None
    for h in range(_NH):
        e = el[h:h + 1, :] + er[:, h:h + 1]
        e = jnp.maximum(e, _SLOPE * e)
        e = jnp.where(mask > 0, e, -1e30)
        m = jnp.max(e, axis=1, keepdims=True)
        p = jnp.exp(e - m) * mask
        d = jnp.sum(p, axis=1, keepdims=True)
        att = p / jnp.maximum(d, 1e-30)
        r = jnp.dot(att, hr, preferred_element_type=f32)
        r = jnp.maximum(r + bg_ref[0:1, h * _F:(h + 1) * _F], 0.0)
        acc = r if acc is None else acc + r

    cat = jnp.concatenate([acc * (1.0 / _NH), hr], axis=1)
    out_ref[...] = jax.lax.dot_general(
        wm8, cat, cdim, preferred_element_type=f32) + bm_col


def kernel(w_iou, b_iou, u_iou, u_f_w, u_f_b, w_type, w_fc, attn_l, attn_r,
           bias_gat, w1, b1, w2, b2, w3, b3, w4, b4,
           x_ast, h0, c0, cfg_type, adj):
    del h0
    f32 = jnp.float32
    g_all = x_ast.shape[0]
    nc = _B * _CL
    chunks = g_all // _CL

    wtf = w_type @ w_fc
    wmt = (w1 @ w2 @ w3 @ w4).T
    bm = ((b1 @ w2 + b2) @ w3 + b3) @ w4 + b4

    hbm = lambda v: pltpu.with_memory_space_constraint(
        v, pltpu.MemorySpace.HBM)
    xb = hbm(jnp.transpose(x_ast, (1, 2, 3, 0)))
    cb = hbm(jnp.transpose(c0, (1, 2, 3, 0)))
    tb = hbm(jnp.transpose(cfg_type, (2, 1, 0)))
    at = jnp.transpose(adj, (1, 0, 2))

    h_root, fsrc = pl.pallas_call(
        _lstm_body(g_all),
        out_shape=[
            jax.ShapeDtypeStruct((_B, g_all, _H), f32),
            jax.ShapeDtypeStruct((_B, g_all, 2 * _H * 2), f32),
        ],
        grid=(_B,),
        in_specs=[
            pl.BlockSpec((1, _NODES, _X, g_all), lambda b: (b, 0, 0, 0)),
            pl.BlockSpec((1, _NODES, _H, g_all), lambda b: (b, 0, 0, 0)),
            pl.BlockSpec((_TD, _B, g_all), lambda b: (0, 0, 0)),
            pl.BlockSpec(w_iou.shape, lambda b: (0, 0)),
            pl.BlockSpec(u_iou.shape, lambda b: (0, 0)),
            pl.BlockSpec(u_f_w.shape, lambda b: (0, 0)),
            pl.BlockSpec((_TD, _NH * _F), lambda b: (0, 0)),
            pl.BlockSpec(b_iou.shape, lambda b: (0, 0)),
            pl.BlockSpec(u_f_b.shape, lambda b: (0, 0)),
        ],
        out_specs=[
            pl.BlockSpec((1, g_all, _H), lambda b: (b, 0, 0)),
            pl.BlockSpec((1, g_all, 2 * _H * 2), lambda b: (b, 0, 0)),
        ],
        compiler_params=pltpu.CompilerParams(
            dimension_semantics=("parallel",)),
        cost_estimate=pl.CostEstimate(
            flops=2_400_000 * g_all, transcendentals=18_000 * g_all,
            bytes_accessed=33_000 * g_all),
    )(xb, cb, tb, w_iou, u_iou, u_f_w, wtf, b_iou, u_f_b)

    out = pl.pallas_call(
        _gat_body,
        out_shape=jax.ShapeDtypeStruct((8, g_all * _B), f32),
        grid=(chunks,),
        in_specs=[
            pl.BlockSpec((_B, _CL, _H), lambda c: (0, c, 0)),
            pl.BlockSpec((_B, _CL, 2 * _H * 2), lambda c: (0, c, 0)),
            pl.BlockSpec((_B, _CL, _B), lambda c: (0, c, 0)),
            pl.BlockSpec(attn_l.shape, lambda c: (0, 0)),
            pl.BlockSpec(attn_r.shape, lambda c: (0, 0)),
            pl.BlockSpec(bias_gat.shape, lambda c: (0, 0)),
            pl.BlockSpec((2, 2 * _H), lambda c: (0, 0)),
            pl.BlockSpec((1, 2), lambda c: (0, 0)),
        ],
        out_specs=pl.BlockSpec((8, nc), lambda c: (0, c)),
        compiler_params=pltpu.CompilerParams(
            dimension_semantics=("parallel",)),
        cost_estimate=pl.CostEstimate(
            flops=600_000 * g_all, transcendentals=3_000 * g_all,
            bytes_accessed=12_000 * g_all),
    )(h_root, fsrc, at, attn_l, attn_r, bias_gat, wmt, bm)

    o = out.reshape(8, chunks, _B, _CL)
    o = jnp.transpose(o, (1, 3, 2, 0))
    return o.reshape(g_all, _B, 8)[:, :, 0:2]
